# Initial kernel scaffold; baseline (speedup 1.0000x reference)
import jax
import jax.numpy as jnp
from jax.experimental import pallas as pl


def kernel(inputs, embeddings):
    raise NotImplementedError("write your pallas kernel here")


# speedup vs baseline: 1.8293x; 1.8293x over previous
import functools

import jax
import jax.numpy as jnp
from jax import lax
from jax.experimental import pallas as pl
from jax.experimental.pallas import tpu as pltpu
from jax.experimental.pallas import tpu_sc as plsc

IN_SIZE = 1000000
OUT_SIZE = 64
BATCH = 16384
HIST = 50

NC, NS = 2, 16
NW = NC * NS
TOTAL = BATCH * HIST
PER_W = TOTAL // NW
STREAM = 128
SPW = 4
CHUNK = STREAM * SPW
NCHUNK = PER_W // CHUNK
ROWS_PER_W = PER_W // STREAM


def _gather_body(idx_hbm, table_hbm, out_hbm, idx_v, rows_v, sem):
    wid = lax.axis_index("s") * NC + lax.axis_index("c")
    pltpu.sync_copy(idx_hbm.at[wid], idx_v)

    out_base = wid * ROWS_PER_W

    def body(j, carry):
        for b in range(SPW):
            pltpu.async_copy(
                table_hbm.at[idx_v.at[j * SPW + b]], rows_v.at[b], sem)
        for b in range(SPW):
            pltpu.make_async_copy(
                table_hbm.at[idx_v.at[j * SPW + b]], rows_v.at[b], sem).wait()
        pltpu.sync_copy(rows_v, out_hbm.at[pl.ds(out_base + j * SPW, SPW)])
        return carry

    lax.fori_loop(0, NCHUNK, body, 0)


@functools.partial(jax.jit, static_argnums=())
def _run(idx, table):
    k = pl.kernel(
        _gather_body,
        out_type=jax.ShapeDtypeStruct((TOTAL // STREAM, STREAM, OUT_SIZE),
                                      jnp.float32),
        mesh=plsc.VectorSubcoreMesh(core_axis_name="c", subcore_axis_name="s"),
        scratch_types=[
            pltpu.VMEM((ROWS_PER_W, STREAM), jnp.int32),
            pltpu.VMEM((SPW, STREAM, OUT_SIZE), jnp.float32),
            pltpu.SemaphoreType.DMA,
        ],
        compiler_params=pltpu.CompilerParams(use_tc_tiling_on_sc=False),
    )
    return k(idx, table)


def kernel(inputs, embeddings):
    idx = inputs.astype(jnp.int32).reshape(NW, ROWS_PER_W, STREAM)
    out = _run(idx, embeddings)
    return out.reshape(BATCH, HIST, OUT_SIZE)


# speedup vs baseline: 1.8739x; 1.0244x over previous
import functools

import jax
import jax.numpy as jnp
from jax import lax
from jax.experimental import pallas as pl
from jax.experimental.pallas import tpu as pltpu
from jax.experimental.pallas import tpu_sc as plsc

IN_SIZE = 1000000
OUT_SIZE = 64
BATCH = 16384
HIST = 50

NC, NS = 2, 16
NW = NC * NS
TOTAL = BATCH * HIST
PER_W = TOTAL // NW
STREAM = 128
SPW = 2
CHUNK = STREAM * SPW
NCHUNK = PER_W // CHUNK
NBUF = 4
ROWS_PER_W = PER_W // STREAM


def _gather_body(idx_hbm, table_hbm, out_hbm, idx_v, rows_v, *sems):
    gsems, ssems = sems[:NBUF], sems[NBUF:]
    wid = lax.axis_index("s") * NC + lax.axis_index("c")
    pltpu.sync_copy(idx_hbm.at[wid], idx_v)
    out_base = wid * ROWS_PER_W

    def issue_gather(j, b):
        for t in range(SPW):
            pltpu.async_copy(
                table_hbm.at[idx_v.at[j * SPW + t]], rows_v.at[b, t], gsems[b])

    def wait_gather(j, b):
        for t in range(SPW):
            pltpu.make_async_copy(
                table_hbm.at[idx_v.at[j * SPW + t]], rows_v.at[b, t],
                gsems[b]).wait()

    def store_descr(j, b):
        return (rows_v.at[b], out_hbm.at[pl.ds(out_base + j * SPW, SPW)],
                ssems[b])

    for b in range(NBUF - 1):
        issue_gather(b, b)

    def body(g, carry):
        for b in range(NBUF):
            j = g * NBUF + b
            bp = (b + NBUF - 1) % NBUF
            wait_gather(j, b)
            pltpu.async_copy(*store_descr(j, b))
            @pl.when(j >= 1)
            def _():
                pltpu.make_async_copy(*store_descr(j - 1, bp)).wait()

            @pl.when(j + NBUF - 1 < NCHUNK)
            def _():
                issue_gather(j + NBUF - 1, bp)
        return carry

    lax.fori_loop(0, NCHUNK // NBUF, body, 0)
    pltpu.make_async_copy(*store_descr(NCHUNK - 1, (NCHUNK - 1) % NBUF)).wait()


@functools.partial(jax.jit, static_argnums=())
def _run(idx, table):
    k = pl.kernel(
        _gather_body,
        out_type=jax.ShapeDtypeStruct((TOTAL // STREAM, STREAM, OUT_SIZE),
                                      jnp.float32),
        mesh=plsc.VectorSubcoreMesh(core_axis_name="c", subcore_axis_name="s"),
        scratch_types=[
            pltpu.VMEM((ROWS_PER_W, STREAM), jnp.int32),
            pltpu.VMEM((NBUF, SPW, STREAM, OUT_SIZE), jnp.float32),
        ] + [pltpu.SemaphoreType.DMA] * (2 * NBUF),
        compiler_params=pltpu.CompilerParams(use_tc_tiling_on_sc=False),
    )
    return k(idx, table)


def kernel(inputs, embeddings):
    idx = inputs.astype(jnp.int32).reshape(NW, ROWS_PER_W, STREAM)
    out = _run(idx, embeddings)
    return out.reshape(BATCH, HIST, OUT_SIZE)
